# baseline (speedup 1.0000x reference)
import functools

import jax
import jax.numpy as jnp
from jax import lax
from jax.experimental import pallas as pl
from jax.experimental.pallas import tpu as pltpu
from jax.experimental.pallas import tpu_sc as plsc

N = 10000
E = 160000
H = 16
G = 64
EPS_BN = 1e-5
SLOPE = 0.01

NC = 2
NS = 16
NW = NC * NS
CH = 128
E_PAD = NW * 40 * CH
ROWS = E_PAD // CH
RPW = ROWS // NW
EPW = E_PAD // NW
BIN = N
NP = N + 16
OPW = N // NS

BE = 2000
GE = E // BE

_f32 = jnp.float32
_X_COLS = (0, 6, 7)
_EA_COLS = (0, 2, 7, 8, 9)


def _col_mask(cols):
    ci = lax.broadcasted_iota(jnp.int32, (1, H), 1)
    m = (ci == cols[0])
    for c in cols[1:]:
        m = m | (ci == c)
    return m.astype(_f32)


def _lrelu(z):
    return jnp.where(z >= 0, z, SLOPE * z)


def _dot(a, b):
    return jnp.dot(a, b, preferred_element_type=_f32)


def _xdot(a, b):
    return jnp.dot(a, b, preferred_element_type=_f32,
                   precision=jax.lax.Precision.HIGHEST)


def _enc2(z, w1, s1, b1, w2, s2, b2):
    h = _lrelu(_dot(z, w1) * s1 + b1)
    return _lrelu(_dot(h, w2) * s2 + b2)


def _ws(shape):
    return pl.BlockSpec(shape, lambda *_: tuple(0 for _ in shape))


def _eb(last=H):
    return pl.BlockSpec((BE, last), lambda i: (i, 0))


def _norm_ab(mask_row, m, var):
    a = mask_row / (jnp.sqrt(var) + 1e-8) + (1.0 - mask_row)
    d = -m * a * mask_row
    return a, d


def _node_body(x_ref, w1, s1, b1, w2, s2, b2, o_ref):
    xv = x_ref[...]
    mask = _col_mask(_X_COLS)
    m = jnp.mean(xv, axis=0, keepdims=True)
    ssq = jnp.sum(xv * xv, axis=0, keepdims=True)
    var = (ssq - N * m * m) / (N - 1)
    a, d = _norm_ab(mask, m, var)
    xn = xv * a + d
    o_ref[...] = _enc2(xn, w1[...], s1[...], b1[...], w2[...], s2[...], b2[...])


def _stats_body(ea_ref, o_ref):
    @pl.when(pl.program_id(0) == 0)
    def _():
        o_ref[...] = jnp.zeros_like(o_ref)

    ev = ea_ref[...]
    o_ref[0:1, :] += jnp.sum(ev, axis=0, keepdims=True)
    o_ref[1:2, :] += jnp.sum(ev * ev, axis=0, keepdims=True)


def _conv_body(ea_ref, st_ref, ew1, es1, eb1, ew2, es2, eb2, xs_ref,
               w1, s1, b1, w2, s2, b2, o_ref):
    HH = H * H
    mask = _col_mask(_EA_COLS)
    st = st_ref[...]
    mu = st[0:1, :] / E
    var = (st[1:2, :] - E * mu * mu) / (E - 1)
    a, d = _norm_ab(mask, mu, var)
    xn = ea_ref[...] * a + d
    ea_enc = _enc2(xn, ew1[...], es1[...], eb1[...], ew2[...], es2[...],
                   eb2[...])
    W = _enc2(ea_enc, w1[...], s1[...], b1[...], w2[...], s2[...],
              b2[...])
    ri = lax.broadcasted_iota(jnp.int32, (H, HH), 0)
    rj = lax.broadcasted_iota(jnp.int32, (H, HH), 1)
    R = (rj // H == ri).astype(_f32)
    si = lax.broadcasted_iota(jnp.int32, (HH, H), 0)
    sj = lax.broadcasted_iota(jnp.int32, (HH, H), 1)
    S = (si % H == sj).astype(_f32)
    xe = _xdot(xs_ref[...], R)
    xeb = xe.astype(jnp.bfloat16).astype(_f32)
    Wb = W.astype(jnp.bfloat16).astype(_f32)
    o_ref[...] = _xdot(xeb * Wb, S)


def _upd_body(agg_ref, x_ref, root, bias, o_ref):
    v = (agg_ref[0] + agg_ref[1] + bias[...] + _dot(x_ref[...], root[...]))
    n = jnp.sqrt(jnp.sum(v * v, axis=1, keepdims=True))
    o_ref[...] = v / jnp.maximum(n, 1e-12)


def _head_body(s_ref, d_ref, wa, wb, s1, b1, w2, s2, b2, w3, b3, o_ref):
    h = _lrelu((_dot(s_ref[...], wa[...]) + _dot(d_ref[...], wb[...]))
               * s1[...] + b1[...])
    h = _lrelu(_dot(h, w2[...]) * s2[...] + b2[...])
    o_ref[...] = _dot(h, w3[...]) + b3[...]


def _graph_body(x_ref, b_ref, gw1, gs1, gb1, gw2, gs2, gb2, dw1, db1, dw2,
                db2, o_ref):
    xv = x_ref[...]
    xg = _enc2(xv, gw1[...], gs1[...], gb1[...], gw2[...], gs2[...], gb2[...])
    bids = b_ref[...]
    gids = lax.broadcasted_iota(jnp.int32, (G, 1), 0)
    oh = (bids == gids).astype(_f32)
    sums = _xdot(oh, xg)
    cnt = jnp.sum(oh, axis=1, keepdims=True)
    mean = sums / jnp.maximum(cnt, 1.0)
    g1 = _lrelu(_dot(mean, dw1[...]) + db1[...])
    o_ref[...] = _dot(g1, dw2[...]) + db2[...]


def _sc_mesh():
    return plsc.VectorSubcoreMesh(core_axis_name="c", subcore_axis_name="s",
                                  num_cores=NC, num_subcores=NS)


def _gather_one(tbl, idx, out, idx_v, rows_v, sem):
    w = lax.axis_index("s") * NC + lax.axis_index("c")
    r0 = w * RPW
    pltpu.sync_copy(idx.at[pl.ds(r0, RPW)], idx_v)
    cps = [pltpu.async_copy(tbl.at[idx_v.at[j]],
                            rows_v.at[pl.ds(j * CH, CH)], sem)
           for j in range(RPW)]
    for cp in cps:
        cp.wait()
    pltpu.sync_copy(rows_v, out.at[pl.ds(r0 * CH, EPW)])


def _gather_body(tbl, idx, out, idx_v, rows_v, sem):
    _gather_one(tbl, idx, out, idx_v, rows_v, sem)


def _gather2_body(tbl, sidx, didx, outs, outd, idx_v, rows_v, sem):
    _gather_one(tbl, sidx, outs, idx_v, rows_v, sem)
    _gather_one(tbl, didx, outd, idx_v, rows_v, sem)


def _scatter_body(m, idx, out, idx_v, rows_v, acc_sh):
    c = lax.axis_index("c")
    s = lax.axis_index("s")
    w = s * NC + c

    def zbody(i, carry):
        rows_v[i] = jnp.zeros((H,), _f32)
        return carry

    lax.fori_loop(0, OPW, zbody, 0)
    pltpu.sync_copy(rows_v.at[pl.ds(0, OPW)], acc_sh.at[pl.ds(s * OPW, OPW)])
    plsc.subcore_barrier()
    pltpu.sync_copy(idx.at[pl.ds(w * RPW, RPW)], idx_v)
    pltpu.sync_copy(m.at[pl.ds(w * EPW, EPW)], rows_v)
    for j in range(RPW):
        pltpu.sync_copy(rows_v.at[pl.ds(j * CH, CH)], acc_sh.at[idx_v.at[j]],
                        add=True)
    plsc.subcore_barrier()
    pltpu.sync_copy(acc_sh.at[pl.ds(s * OPW, OPW)], rows_v.at[pl.ds(0, OPW)])
    pltpu.sync_copy(rows_v.at[pl.ds(0, OPW)], out.at[c, pl.ds(s * OPW, OPW)])


def _make_sc_calls():
    params = pltpu.CompilerParams(use_tc_tiling_on_sc=False)
    gather = pl.kernel(
        _gather_body,
        out_type=jax.ShapeDtypeStruct((E_PAD, H), _f32),
        mesh=_sc_mesh(),
        compiler_params=params,
        scratch_types=[pltpu.VMEM((RPW, CH), jnp.int32),
                       pltpu.VMEM((EPW, H), _f32),
                       pltpu.SemaphoreType.DMA],
    )
    gather2 = pl.kernel(
        _gather2_body,
        out_type=(jax.ShapeDtypeStruct((E_PAD, H), _f32),
                  jax.ShapeDtypeStruct((E_PAD, H), _f32)),
        mesh=_sc_mesh(),
        compiler_params=params,
        scratch_types=[pltpu.VMEM((RPW, CH), jnp.int32),
                       pltpu.VMEM((EPW, H), _f32),
                       pltpu.SemaphoreType.DMA],
    )
    scatter = pl.kernel(
        _scatter_body,
        out_type=jax.ShapeDtypeStruct((NC, N, H), _f32),
        mesh=_sc_mesh(),
        compiler_params=params,
        scratch_types=[pltpu.VMEM((RPW, CH), jnp.int32),
                       pltpu.VMEM((EPW, H), _f32),
                       pltpu.VMEM_SHARED((NP, H), _f32)],
    )
    return gather, gather2, scatter


def _node_call(x, ww):
    return pl.pallas_call(
        _node_body, grid=(1,),
        in_specs=[_ws((N, H)), _ws((H, H)), _ws((1, H)), _ws((1, H)),
                  _ws((H, H)), _ws((1, H)), _ws((1, H))],
        out_specs=_ws((N, H)),
        out_shape=jax.ShapeDtypeStruct((N, H), _f32),
    )(x, *ww)


def _stats_call(ea):
    return pl.pallas_call(
        _stats_body, grid=(GE,),
        in_specs=[_eb()],
        out_specs=_ws((2, H)),
        out_shape=jax.ShapeDtypeStruct((2, H), _f32),
    )(ea)


def _conv_call(ea, st, ew, xs, cw):
    return pl.pallas_call(
        _conv_body, grid=(GE,),
        in_specs=[_eb(), _ws((2, H)),
                  _ws((H, H)), _ws((1, H)), _ws((1, H)),
                  _ws((H, H)), _ws((1, H)), _ws((1, H)),
                  _eb(),
                  _ws((H, H)), _ws((1, H)), _ws((1, H)),
                  _ws((H, H * H)), _ws((1, H * H)), _ws((1, H * H))],
        out_specs=_eb(),
        out_shape=jax.ShapeDtypeStruct((E_PAD, H), _f32),
    )(ea, st, *ew, xs, *cw)


def _upd_call(agg, x, root, bias):
    return pl.pallas_call(
        _upd_body, grid=(1,),
        in_specs=[pl.BlockSpec((NC, N, H), lambda i: (0, 0, 0)),
                  _ws((N, H)), _ws((H, H)), _ws((1, H))],
        out_specs=_ws((N, H)),
        out_shape=jax.ShapeDtypeStruct((N, H), _f32),
    )(agg, x, root, bias)


def _head_call(xs, xd, wa, wb, s1, b1, w2, s2, b2, w3, b3):
    return pl.pallas_call(
        _head_body, grid=(GE,),
        in_specs=[_eb(), _eb(), _ws((H, H)), _ws((H, H)), _ws((1, H)),
                  _ws((1, H)), _ws((H, H)), _ws((1, H)), _ws((1, H)),
                  _ws((H, 1)), _ws((1, 1))],
        out_specs=_eb(1),
        out_shape=jax.ShapeDtypeStruct((E, 1), _f32),
    )(xs, xd, wa, wb, s1, b1, w2, s2, b2, w3, b3)


def _graph_call(x3, bat, gw, dw1, db1, dw2, db2):
    return pl.pallas_call(
        _graph_body, grid=(1,),
        in_specs=[_ws((N, H)), _ws((1, N)),
                  _ws((H, H)), _ws((1, H)), _ws((1, H)),
                  _ws((H, H)), _ws((1, H)), _ws((1, H)),
                  _ws((H, H)), _ws((1, H)), _ws((H, H)), _ws((1, H))],
        out_specs=_ws((G, H)),
        out_shape=jax.ShapeDtypeStruct((G, H), _f32),
    )(x3, bat, *gw, dw1, db1, dw2, db2)


def _fold(p, pre):
    s1 = p[pre + '_g1'] / jnp.sqrt(1.0 + EPS_BN)
    b1 = (p[pre + '_b1'] * s1 + p[pre + '_be1'])[None, :]
    s2 = p[pre + '_g2'] / jnp.sqrt(1.0 + EPS_BN)
    b2 = (p[pre + '_b2'] * s2 + p[pre + '_be2'])[None, :]
    return (p[pre + '_w1'], s1[None, :], b1,
            p[pre + '_w2'], s2[None, :], b2)


def kernel(x, edge_index, edge_attr, batch, params):
    p = params
    src = edge_index[0]
    dst = edge_index[1]
    padn = E_PAD - E
    src2d = jnp.concatenate(
        [src, jnp.zeros((padn,), jnp.int32)]).reshape(ROWS, CH)
    dst2d = jnp.concatenate(
        [dst, jnp.full((padn,), BIN, jnp.int32)]).reshape(ROWS, CH)
    bat = batch.reshape(1, N)

    gather, gather2, scatter = _make_sc_calls()

    ne = _fold(p, 'ne')
    ew = _fold(p, 'ee')
    c1 = _fold(p, 'c1')
    c2 = _fold(p, 'c2')
    rw1, rs1, rb1, rw2, rs2, rb2 = _fold(p, 'r')
    gw = _fold(p, 'g')

    x_enc = _node_call(x, ne)
    ea_st = _stats_call(edge_attr)

    xs1 = gather(x_enc, src2d)
    m1 = _conv_call(edge_attr, ea_st, ew, xs1, c1)
    agg1 = scatter(m1, dst2d)
    x2 = _upd_call(agg1, x_enc, p['c1_root'], p['c1_bias'][None, :])

    xs2 = gather(x2, src2d)
    m2 = _conv_call(edge_attr, ea_st, ew, xs2, c2)
    agg2 = scatter(m2, dst2d)
    x3 = _upd_call(agg2, x2, p['c2_root'], p['c2_bias'][None, :])

    xs3, xd3 = gather2(x3, src2d, dst2d)
    scores = _head_call(xs3, xd3, rw1[:H], rw1[H:], rs1, rb1, rw2, rs2, rb2,
                        p['r_w3'], p['r_b3'][None, :])
    gemb = _graph_call(x3, bat, gw,
                       p['d_w1'], p['d_b1'][None, :],
                       p['d_w2'], p['d_b2'][None, :])
    return (scores, gemb)


# speedup vs baseline: 1.5944x; 1.5944x over previous
import functools

import jax
import jax.numpy as jnp
from jax import lax
from jax.experimental import pallas as pl
from jax.experimental.pallas import tpu as pltpu
from jax.experimental.pallas import tpu_sc as plsc

N = 10000
E = 160000
H = 16
G = 64
EPS_BN = 1e-5
SLOPE = 0.01

NC = 2
NS = 16
NW = NC * NS
CH = 128
E_PAD = NW * 40 * CH
ROWS = E_PAD // CH
RPW = ROWS // NW
EPW = E_PAD // NW
BIN = N
NP = N + 16
OPW = N // NS

BE = 2000
GE = E // BE

_f32 = jnp.float32
_X_COLS = (0, 6, 7)
_EA_COLS = (0, 2, 7, 8, 9)


def _col_mask(cols):
    ci = lax.broadcasted_iota(jnp.int32, (1, H), 1)
    m = (ci == cols[0])
    for c in cols[1:]:
        m = m | (ci == c)
    return m.astype(_f32)


def _lrelu(z):
    return jnp.where(z >= 0, z, SLOPE * z)


def _dot(a, b):
    return jnp.dot(a, b, preferred_element_type=_f32)


def _xdot(a, b):
    return jnp.dot(a, b, preferred_element_type=_f32,
                   precision=jax.lax.Precision.HIGHEST)


def _enc2(z, w1, s1, b1, w2, s2, b2):
    h = _lrelu(_dot(z, w1) * s1 + b1)
    return _lrelu(_dot(h, w2) * s2 + b2)


def _ws(shape):
    return pl.BlockSpec(shape, lambda *_: tuple(0 for _ in shape))


def _eb(last=H):
    return pl.BlockSpec((BE, last), lambda i: (i, 0))


def _norm_ab(mask_row, m, var):
    a = mask_row / (jnp.sqrt(var) + 1e-8) + (1.0 - mask_row)
    d = -m * a * mask_row
    return a, d


def _node_body(x_ref, w1, s1, b1, w2, s2, b2, o_ref):
    xv = x_ref[...]
    mask = _col_mask(_X_COLS)
    m = jnp.mean(xv, axis=0, keepdims=True)
    ssq = jnp.sum(xv * xv, axis=0, keepdims=True)
    var = (ssq - N * m * m) / (N - 1)
    a, d = _norm_ab(mask, m, var)
    xn = xv * a + d
    o_ref[...] = _enc2(xn, w1[...], s1[...], b1[...], w2[...], s2[...], b2[...])


def _stats_body(ea_ref, o_ref):
    @pl.when(pl.program_id(0) == 0)
    def _():
        o_ref[...] = jnp.zeros_like(o_ref)

    ev = ea_ref[...]
    o_ref[0:1, :] += jnp.sum(ev, axis=0, keepdims=True)
    o_ref[1:2, :] += jnp.sum(ev * ev, axis=0, keepdims=True)


def _conv_body(ea_ref, st_ref, ew1, es1, eb1, ew2, es2, eb2, xs_ref,
               w1, s1, b1, w2, s2, b2, o_ref):
    HH = H * H
    mask = _col_mask(_EA_COLS)
    st = st_ref[...]
    mu = st[0:1, :] / E
    var = (st[1:2, :] - E * mu * mu) / (E - 1)
    a, d = _norm_ab(mask, mu, var)
    xn = ea_ref[...] * a + d
    ea_enc = _enc2(xn, ew1[...], es1[...], eb1[...], ew2[...], es2[...],
                   eb2[...])
    W = _enc2(ea_enc, w1[...], s1[...], b1[...], w2[...], s2[...],
              b2[...])
    ri = lax.broadcasted_iota(jnp.int32, (H, HH), 0)
    rj = lax.broadcasted_iota(jnp.int32, (H, HH), 1)
    R = (rj // H == ri).astype(_f32)
    si = lax.broadcasted_iota(jnp.int32, (HH, H), 0)
    sj = lax.broadcasted_iota(jnp.int32, (HH, H), 1)
    S = (si % H == sj).astype(_f32)
    bf = jnp.bfloat16
    xeb = jnp.dot(xs_ref[...].astype(bf), R.astype(bf),
                  preferred_element_type=_f32)
    P = xeb * W.astype(bf).astype(_f32)
    hi = P.astype(bf)
    lo = (P - hi.astype(_f32)).astype(bf)
    Sb = S.astype(bf)
    o_ref[...] = (jnp.dot(hi, Sb, preferred_element_type=_f32)
                  + jnp.dot(lo, Sb, preferred_element_type=_f32))


def _upd_body(agg_ref, x_ref, root, bias, o_ref):
    v = (agg_ref[0] + agg_ref[1] + bias[...] + _dot(x_ref[...], root[...]))
    n = jnp.sqrt(jnp.sum(v * v, axis=1, keepdims=True))
    o_ref[...] = v / jnp.maximum(n, 1e-12)


def _head_body(s_ref, d_ref, wa, wb, s1, b1, w2, s2, b2, w3, b3, o_ref):
    h = _lrelu((_dot(s_ref[...], wa[...]) + _dot(d_ref[...], wb[...]))
               * s1[...] + b1[...])
    h = _lrelu(_dot(h, w2[...]) * s2[...] + b2[...])
    o_ref[...] = _dot(h, w3[...]) + b3[...]


def _graph_body(x_ref, b_ref, gw1, gs1, gb1, gw2, gs2, gb2, dw1, db1, dw2,
                db2, o_ref):
    xv = x_ref[...]
    xg = _enc2(xv, gw1[...], gs1[...], gb1[...], gw2[...], gs2[...], gb2[...])
    bids = b_ref[...]
    gids = lax.broadcasted_iota(jnp.int32, (G, 1), 0)
    oh = (bids == gids).astype(jnp.bfloat16)
    xh = xg.astype(jnp.bfloat16)
    xl = (xg - xh.astype(_f32)).astype(jnp.bfloat16)
    sums = (jnp.dot(oh, xh, preferred_element_type=_f32)
            + jnp.dot(oh, xl, preferred_element_type=_f32))
    cnt = jnp.sum(oh.astype(_f32), axis=1, keepdims=True)
    mean = sums / jnp.maximum(cnt, 1.0)
    g1 = _lrelu(_dot(mean, dw1[...]) + db1[...])
    o_ref[...] = _dot(g1, dw2[...]) + db2[...]


def _sc_mesh():
    return plsc.VectorSubcoreMesh(core_axis_name="c", subcore_axis_name="s",
                                  num_cores=NC, num_subcores=NS)


def _gather_one(tbl, idx, out, idx_v, rows_v, sem):
    w = lax.axis_index("s") * NC + lax.axis_index("c")
    r0 = w * RPW
    pltpu.sync_copy(idx.at[pl.ds(r0, RPW)], idx_v)
    cps = [pltpu.async_copy(tbl.at[idx_v.at[j]],
                            rows_v.at[pl.ds(j * CH, CH)], sem)
           for j in range(RPW)]
    for cp in cps:
        cp.wait()
    pltpu.sync_copy(rows_v, out.at[pl.ds(r0 * CH, EPW)])


def _gather_body(tbl, idx, out, idx_v, rows_v, sem):
    _gather_one(tbl, idx, out, idx_v, rows_v, sem)


def _gather2_body(tbl, sidx, didx, outs, outd, idx_v, rows_v, sem):
    _gather_one(tbl, sidx, outs, idx_v, rows_v, sem)
    _gather_one(tbl, didx, outd, idx_v, rows_v, sem)


def _scatter_body(m, idx, out, idx_v, rows_v, acc_sh):
    c = lax.axis_index("c")
    s = lax.axis_index("s")
    w = s * NC + c

    def zbody(i, carry):
        rows_v[i] = jnp.zeros((H,), _f32)
        return carry

    lax.fori_loop(0, OPW, zbody, 0)
    pltpu.sync_copy(rows_v.at[pl.ds(0, OPW)], acc_sh.at[pl.ds(s * OPW, OPW)])
    plsc.subcore_barrier()
    pltpu.sync_copy(idx.at[pl.ds(w * RPW, RPW)], idx_v)
    pltpu.sync_copy(m.at[pl.ds(w * EPW, EPW)], rows_v)
    for j in range(RPW):
        pltpu.sync_copy(rows_v.at[pl.ds(j * CH, CH)], acc_sh.at[idx_v.at[j]],
                        add=True)
    plsc.subcore_barrier()
    pltpu.sync_copy(acc_sh.at[pl.ds(s * OPW, OPW)], rows_v.at[pl.ds(0, OPW)])
    pltpu.sync_copy(rows_v.at[pl.ds(0, OPW)], out.at[c, pl.ds(s * OPW, OPW)])


def _make_sc_calls():
    params = pltpu.CompilerParams(use_tc_tiling_on_sc=False)
    gather = pl.kernel(
        _gather_body,
        out_type=jax.ShapeDtypeStruct((E_PAD, H), _f32),
        mesh=_sc_mesh(),
        compiler_params=params,
        scratch_types=[pltpu.VMEM((RPW, CH), jnp.int32),
                       pltpu.VMEM((EPW, H), _f32),
                       pltpu.SemaphoreType.DMA],
    )
    gather2 = pl.kernel(
        _gather2_body,
        out_type=(jax.ShapeDtypeStruct((E_PAD, H), _f32),
                  jax.ShapeDtypeStruct((E_PAD, H), _f32)),
        mesh=_sc_mesh(),
        compiler_params=params,
        scratch_types=[pltpu.VMEM((RPW, CH), jnp.int32),
                       pltpu.VMEM((EPW, H), _f32),
                       pltpu.SemaphoreType.DMA],
    )
    scatter = pl.kernel(
        _scatter_body,
        out_type=jax.ShapeDtypeStruct((NC, N, H), _f32),
        mesh=_sc_mesh(),
        compiler_params=params,
        scratch_types=[pltpu.VMEM((RPW, CH), jnp.int32),
                       pltpu.VMEM((EPW, H), _f32),
                       pltpu.VMEM_SHARED((NP, H), _f32)],
    )
    return gather, gather2, scatter


def _node_call(x, ww):
    return pl.pallas_call(
        _node_body, grid=(1,),
        in_specs=[_ws((N, H)), _ws((H, H)), _ws((1, H)), _ws((1, H)),
                  _ws((H, H)), _ws((1, H)), _ws((1, H))],
        out_specs=_ws((N, H)),
        out_shape=jax.ShapeDtypeStruct((N, H), _f32),
    )(x, *ww)


def _stats_call(ea):
    return pl.pallas_call(
        _stats_body, grid=(GE,),
        in_specs=[_eb()],
        out_specs=_ws((2, H)),
        out_shape=jax.ShapeDtypeStruct((2, H), _f32),
    )(ea)


def _conv_call(ea, st, ew, xs, cw):
    return pl.pallas_call(
        _conv_body, grid=(GE,),
        in_specs=[_eb(), _ws((2, H)),
                  _ws((H, H)), _ws((1, H)), _ws((1, H)),
                  _ws((H, H)), _ws((1, H)), _ws((1, H)),
                  _eb(),
                  _ws((H, H)), _ws((1, H)), _ws((1, H)),
                  _ws((H, H * H)), _ws((1, H * H)), _ws((1, H * H))],
        out_specs=_eb(),
        out_shape=jax.ShapeDtypeStruct((E_PAD, H), _f32),
    )(ea, st, *ew, xs, *cw)


def _upd_call(agg, x, root, bias):
    return pl.pallas_call(
        _upd_body, grid=(1,),
        in_specs=[pl.BlockSpec((NC, N, H), lambda i: (0, 0, 0)),
                  _ws((N, H)), _ws((H, H)), _ws((1, H))],
        out_specs=_ws((N, H)),
        out_shape=jax.ShapeDtypeStruct((N, H), _f32),
    )(agg, x, root, bias)


def _head_call(xs, xd, wa, wb, s1, b1, w2, s2, b2, w3, b3):
    return pl.pallas_call(
        _head_body, grid=(GE,),
        in_specs=[_eb(), _eb(), _ws((H, H)), _ws((H, H)), _ws((1, H)),
                  _ws((1, H)), _ws((H, H)), _ws((1, H)), _ws((1, H)),
                  _ws((H, 1)), _ws((1, 1))],
        out_specs=_eb(1),
        out_shape=jax.ShapeDtypeStruct((E, 1), _f32),
    )(xs, xd, wa, wb, s1, b1, w2, s2, b2, w3, b3)


def _graph_call(x3, bat, gw, dw1, db1, dw2, db2):
    return pl.pallas_call(
        _graph_body, grid=(1,),
        in_specs=[_ws((N, H)), _ws((1, N)),
                  _ws((H, H)), _ws((1, H)), _ws((1, H)),
                  _ws((H, H)), _ws((1, H)), _ws((1, H)),
                  _ws((H, H)), _ws((1, H)), _ws((H, H)), _ws((1, H))],
        out_specs=_ws((G, H)),
        out_shape=jax.ShapeDtypeStruct((G, H), _f32),
    )(x3, bat, *gw, dw1, db1, dw2, db2)


def _fold(p, pre):
    s1 = p[pre + '_g1'] / jnp.sqrt(1.0 + EPS_BN)
    b1 = (p[pre + '_b1'] * s1 + p[pre + '_be1'])[None, :]
    s2 = p[pre + '_g2'] / jnp.sqrt(1.0 + EPS_BN)
    b2 = (p[pre + '_b2'] * s2 + p[pre + '_be2'])[None, :]
    return (p[pre + '_w1'], s1[None, :], b1,
            p[pre + '_w2'], s2[None, :], b2)


def kernel(x, edge_index, edge_attr, batch, params):
    p = params
    src = edge_index[0]
    dst = edge_index[1]
    padn = E_PAD - E
    src2d = jnp.concatenate(
        [src, jnp.zeros((padn,), jnp.int32)]).reshape(ROWS, CH)
    dst2d = jnp.concatenate(
        [dst, jnp.full((padn,), BIN, jnp.int32)]).reshape(ROWS, CH)
    bat = batch.reshape(1, N)

    gather, gather2, scatter = _make_sc_calls()

    ne = _fold(p, 'ne')
    ew = _fold(p, 'ee')
    c1 = _fold(p, 'c1')
    c2 = _fold(p, 'c2')
    rw1, rs1, rb1, rw2, rs2, rb2 = _fold(p, 'r')
    gw = _fold(p, 'g')

    x_enc = _node_call(x, ne)
    ea_st = _stats_call(edge_attr)

    xs1 = gather(x_enc, src2d)
    m1 = _conv_call(edge_attr, ea_st, ew, xs1, c1)
    agg1 = scatter(m1, dst2d)
    x2 = _upd_call(agg1, x_enc, p['c1_root'], p['c1_bias'][None, :])

    xs2 = gather(x2, src2d)
    m2 = _conv_call(edge_attr, ea_st, ew, xs2, c2)
    agg2 = scatter(m2, dst2d)
    x3 = _upd_call(agg2, x2, p['c2_root'], p['c2_bias'][None, :])

    xs3, xd3 = gather2(x3, src2d, dst2d)
    scores = _head_call(xs3, xd3, rw1[:H], rw1[H:], rs1, rb1, rw2, rs2, rb2,
                        p['r_w3'], p['r_b3'][None, :])
    gemb = _graph_call(x3, bat, gw,
                       p['d_w1'], p['d_b1'][None, :],
                       p['d_w2'], p['d_b2'][None, :])
    return (scores, gemb)


# speedup vs baseline: 1.7069x; 1.0706x over previous
import functools

import jax
import jax.numpy as jnp
from jax import lax
from jax.experimental import pallas as pl
from jax.experimental.pallas import tpu as pltpu
from jax.experimental.pallas import tpu_sc as plsc

N = 10000
E = 160000
H = 16
G = 64
EPS_BN = 1e-5
SLOPE = 0.01

NC = 2
NS = 16
NW = NC * NS
CH = 128
E_PAD = NW * 40 * CH
ROWS = E_PAD // CH
RPW = ROWS // NW
EPW = E_PAD // NW
BIN = N
NP = N + 16
OPW = N // NS

BE = 8000
GE = E // BE

_f32 = jnp.float32
_X_COLS = (0, 6, 7)
_EA_COLS = (0, 2, 7, 8, 9)


def _col_mask(cols):
    ci = lax.broadcasted_iota(jnp.int32, (1, H), 1)
    m = (ci == cols[0])
    for c in cols[1:]:
        m = m | (ci == c)
    return m.astype(_f32)


def _lrelu(z):
    return jnp.where(z >= 0, z, SLOPE * z)


def _dot(a, b):
    return jnp.dot(a, b, preferred_element_type=_f32)


def _xdot(a, b):
    return jnp.dot(a, b, preferred_element_type=_f32,
                   precision=jax.lax.Precision.HIGHEST)


def _enc2(z, w1, s1, b1, w2, s2, b2):
    h = _lrelu(_dot(z, w1) * s1 + b1)
    return _lrelu(_dot(h, w2) * s2 + b2)


def _ws(shape):
    return pl.BlockSpec(shape, lambda *_: tuple(0 for _ in shape))


def _eb(last=H):
    return pl.BlockSpec((BE, last), lambda i: (i, 0))


def _norm_ab(mask_row, m, var):
    a = mask_row / (jnp.sqrt(var) + 1e-8) + (1.0 - mask_row)
    d = -m * a * mask_row
    return a, d


def _node_body(x_ref, w1, s1, b1, w2, s2, b2, o_ref):
    xv = x_ref[...]
    mask = _col_mask(_X_COLS)
    m = jnp.mean(xv, axis=0, keepdims=True)
    ssq = jnp.sum(xv * xv, axis=0, keepdims=True)
    var = (ssq - N * m * m) / (N - 1)
    a, d = _norm_ab(mask, m, var)
    xn = xv * a + d
    o_ref[...] = _enc2(xn, w1[...], s1[...], b1[...], w2[...], s2[...], b2[...])


def _stats_body(ea_ref, o_ref):
    @pl.when(pl.program_id(0) == 0)
    def _():
        o_ref[...] = jnp.zeros_like(o_ref)

    ev = ea_ref[...]
    o_ref[0:1, :] += jnp.sum(ev, axis=0, keepdims=True)
    o_ref[1:2, :] += jnp.sum(ev * ev, axis=0, keepdims=True)


def _conv_body(ea_ref, st_ref, ew1, es1, eb1, ew2, es2, eb2, xs_ref,
               w1, s1, b1, w2, s2, b2, o_ref):
    HH = H * H
    mask = _col_mask(_EA_COLS)
    st = st_ref[...]
    mu = st[0:1, :] / E
    var = (st[1:2, :] - E * mu * mu) / (E - 1)
    a, d = _norm_ab(mask, mu, var)
    xn = ea_ref[...] * a + d
    ea_enc = _enc2(xn, ew1[...], es1[...], eb1[...], ew2[...], es2[...],
                   eb2[...])
    W = _enc2(ea_enc, w1[...], s1[...], b1[...], w2[...], s2[...],
              b2[...])
    ri = lax.broadcasted_iota(jnp.int32, (H, HH), 0)
    rj = lax.broadcasted_iota(jnp.int32, (H, HH), 1)
    R = (rj // H == ri).astype(_f32)
    si = lax.broadcasted_iota(jnp.int32, (HH, H), 0)
    sj = lax.broadcasted_iota(jnp.int32, (HH, H), 1)
    S = (si % H == sj).astype(_f32)
    bf = jnp.bfloat16
    xeb = jnp.dot(xs_ref[...].astype(bf), R.astype(bf),
                  preferred_element_type=_f32)
    P = xeb * W.astype(bf).astype(_f32)
    hi = P.astype(bf)
    lo = (P - hi.astype(_f32)).astype(bf)
    Sb = S.astype(bf)
    o_ref[...] = (jnp.dot(hi, Sb, preferred_element_type=_f32)
                  + jnp.dot(lo, Sb, preferred_element_type=_f32))


def _upd_body(agg_ref, x_ref, root, bias, o_ref):
    v = (agg_ref[0] + agg_ref[1] + bias[...] + _dot(x_ref[...], root[...]))
    n = jnp.sqrt(jnp.sum(v * v, axis=1, keepdims=True))
    o_ref[...] = v / jnp.maximum(n, 1e-12)


def _head_body(s_ref, d_ref, wa, wb, s1, b1, w2, s2, b2, w3, b3, o_ref):
    h = _lrelu((_dot(s_ref[...], wa[...]) + _dot(d_ref[...], wb[...]))
               * s1[...] + b1[...])
    h = _lrelu(_dot(h, w2[...]) * s2[...] + b2[...])
    o_ref[...] = _dot(h, w3[...]) + b3[...]


def _graph_body(x_ref, b_ref, gw1, gs1, gb1, gw2, gs2, gb2, dw1, db1, dw2,
                db2, o_ref):
    xv = x_ref[...]
    xg = _enc2(xv, gw1[...], gs1[...], gb1[...], gw2[...], gs2[...], gb2[...])
    bids = b_ref[...]
    gids = lax.broadcasted_iota(jnp.int32, (G, 1), 0)
    oh = (bids == gids).astype(jnp.bfloat16)
    xh = xg.astype(jnp.bfloat16)
    xl = (xg - xh.astype(_f32)).astype(jnp.bfloat16)
    sums = (jnp.dot(oh, xh, preferred_element_type=_f32)
            + jnp.dot(oh, xl, preferred_element_type=_f32))
    cnt = jnp.sum(oh.astype(_f32), axis=1, keepdims=True)
    mean = sums / jnp.maximum(cnt, 1.0)
    g1 = _lrelu(_dot(mean, dw1[...]) + db1[...])
    o_ref[...] = _dot(g1, dw2[...]) + db2[...]


def _sc_mesh():
    return plsc.VectorSubcoreMesh(core_axis_name="c", subcore_axis_name="s",
                                  num_cores=NC, num_subcores=NS)


def _gather_one(tbl, idx, out, idx_v, rows_v, sem):
    w = lax.axis_index("s") * NC + lax.axis_index("c")
    r0 = w * RPW
    pltpu.sync_copy(idx.at[pl.ds(r0, RPW)], idx_v)
    cps = [pltpu.async_copy(tbl.at[idx_v.at[j]],
                            rows_v.at[pl.ds(j * CH, CH)], sem)
           for j in range(RPW)]
    for cp in cps:
        cp.wait()
    pltpu.sync_copy(rows_v, out.at[pl.ds(r0 * CH, EPW)])


def _gather_body(tbl, idx, out, idx_v, rows_v, sem):
    _gather_one(tbl, idx, out, idx_v, rows_v, sem)


def _gather2_body(tbl, sidx, didx, outs, outd, idx_v, rows_v, sem):
    _gather_one(tbl, sidx, outs, idx_v, rows_v, sem)
    _gather_one(tbl, didx, outd, idx_v, rows_v, sem)


def _scatter_body(m, idx, out, idx_v, rows_v, acc_sh):
    c = lax.axis_index("c")
    s = lax.axis_index("s")
    w = s * NC + c

    def zbody(i, carry):
        rows_v[i] = jnp.zeros((H,), _f32)
        return carry

    lax.fori_loop(0, OPW, zbody, 0)
    pltpu.sync_copy(rows_v.at[pl.ds(0, OPW)], acc_sh.at[pl.ds(s * OPW, OPW)])
    plsc.subcore_barrier()
    pltpu.sync_copy(idx.at[pl.ds(w * RPW, RPW)], idx_v)
    pltpu.sync_copy(m.at[pl.ds(w * EPW, EPW)], rows_v)
    for j in range(RPW):
        pltpu.sync_copy(rows_v.at[pl.ds(j * CH, CH)], acc_sh.at[idx_v.at[j]],
                        add=True)
    plsc.subcore_barrier()
    pltpu.sync_copy(acc_sh.at[pl.ds(s * OPW, OPW)], rows_v.at[pl.ds(0, OPW)])
    pltpu.sync_copy(rows_v.at[pl.ds(0, OPW)], out.at[c, pl.ds(s * OPW, OPW)])


def _make_sc_calls():
    params = pltpu.CompilerParams(use_tc_tiling_on_sc=False)
    gather = pl.kernel(
        _gather_body,
        out_type=jax.ShapeDtypeStruct((E_PAD, H), _f32),
        mesh=_sc_mesh(),
        compiler_params=params,
        scratch_types=[pltpu.VMEM((RPW, CH), jnp.int32),
                       pltpu.VMEM((EPW, H), _f32),
                       pltpu.SemaphoreType.DMA],
    )
    gather2 = pl.kernel(
        _gather2_body,
        out_type=(jax.ShapeDtypeStruct((E_PAD, H), _f32),
                  jax.ShapeDtypeStruct((E_PAD, H), _f32)),
        mesh=_sc_mesh(),
        compiler_params=params,
        scratch_types=[pltpu.VMEM((RPW, CH), jnp.int32),
                       pltpu.VMEM((EPW, H), _f32),
                       pltpu.SemaphoreType.DMA],
    )
    scatter = pl.kernel(
        _scatter_body,
        out_type=jax.ShapeDtypeStruct((NC, N, H), _f32),
        mesh=_sc_mesh(),
        compiler_params=params,
        scratch_types=[pltpu.VMEM((RPW, CH), jnp.int32),
                       pltpu.VMEM((EPW, H), _f32),
                       pltpu.VMEM_SHARED((NP, H), _f32)],
    )
    return gather, gather2, scatter


def _node_call(x, ww):
    return pl.pallas_call(
        _node_body, grid=(1,),
        in_specs=[_ws((N, H)), _ws((H, H)), _ws((1, H)), _ws((1, H)),
                  _ws((H, H)), _ws((1, H)), _ws((1, H))],
        out_specs=_ws((N, H)),
        out_shape=jax.ShapeDtypeStruct((N, H), _f32),
    )(x, *ww)


def _stats_call(ea):
    return pl.pallas_call(
        _stats_body, grid=(GE,),
        in_specs=[_eb()],
        out_specs=_ws((2, H)),
        out_shape=jax.ShapeDtypeStruct((2, H), _f32),
    )(ea)


def _conv_call(ea, st, ew, xs, cw):
    return pl.pallas_call(
        _conv_body, grid=(GE,),
        in_specs=[_eb(), _ws((2, H)),
                  _ws((H, H)), _ws((1, H)), _ws((1, H)),
                  _ws((H, H)), _ws((1, H)), _ws((1, H)),
                  _eb(),
                  _ws((H, H)), _ws((1, H)), _ws((1, H)),
                  _ws((H, H * H)), _ws((1, H * H)), _ws((1, H * H))],
        out_specs=_eb(),
        out_shape=jax.ShapeDtypeStruct((E_PAD, H), _f32),
    )(ea, st, *ew, xs, *cw)


def _upd_call(agg, x, root, bias):
    return pl.pallas_call(
        _upd_body, grid=(1,),
        in_specs=[pl.BlockSpec((NC, N, H), lambda i: (0, 0, 0)),
                  _ws((N, H)), _ws((H, H)), _ws((1, H))],
        out_specs=_ws((N, H)),
        out_shape=jax.ShapeDtypeStruct((N, H), _f32),
    )(agg, x, root, bias)


def _head_call(xs, xd, wa, wb, s1, b1, w2, s2, b2, w3, b3):
    return pl.pallas_call(
        _head_body, grid=(GE,),
        in_specs=[_eb(), _eb(), _ws((H, H)), _ws((H, H)), _ws((1, H)),
                  _ws((1, H)), _ws((H, H)), _ws((1, H)), _ws((1, H)),
                  _ws((H, 1)), _ws((1, 1))],
        out_specs=_eb(1),
        out_shape=jax.ShapeDtypeStruct((E, 1), _f32),
    )(xs, xd, wa, wb, s1, b1, w2, s2, b2, w3, b3)


def _graph_call(x3, bat, gw, dw1, db1, dw2, db2):
    return pl.pallas_call(
        _graph_body, grid=(1,),
        in_specs=[_ws((N, H)), _ws((1, N)),
                  _ws((H, H)), _ws((1, H)), _ws((1, H)),
                  _ws((H, H)), _ws((1, H)), _ws((1, H)),
                  _ws((H, H)), _ws((1, H)), _ws((H, H)), _ws((1, H))],
        out_specs=_ws((G, H)),
        out_shape=jax.ShapeDtypeStruct((G, H), _f32),
    )(x3, bat, *gw, dw1, db1, dw2, db2)


def _fold(p, pre):
    s1 = p[pre + '_g1'] / jnp.sqrt(1.0 + EPS_BN)
    b1 = (p[pre + '_b1'] * s1 + p[pre + '_be1'])[None, :]
    s2 = p[pre + '_g2'] / jnp.sqrt(1.0 + EPS_BN)
    b2 = (p[pre + '_b2'] * s2 + p[pre + '_be2'])[None, :]
    return (p[pre + '_w1'], s1[None, :], b1,
            p[pre + '_w2'], s2[None, :], b2)


def kernel(x, edge_index, edge_attr, batch, params):
    p = params
    src = edge_index[0]
    dst = edge_index[1]
    padn = E_PAD - E
    src2d = jnp.concatenate(
        [src, jnp.zeros((padn,), jnp.int32)]).reshape(ROWS, CH)
    dst2d = jnp.concatenate(
        [dst, jnp.full((padn,), BIN, jnp.int32)]).reshape(ROWS, CH)
    bat = batch.reshape(1, N)

    gather, gather2, scatter = _make_sc_calls()

    ne = _fold(p, 'ne')
    ew = _fold(p, 'ee')
    c1 = _fold(p, 'c1')
    c2 = _fold(p, 'c2')
    rw1, rs1, rb1, rw2, rs2, rb2 = _fold(p, 'r')
    gw = _fold(p, 'g')

    x_enc = _node_call(x, ne)
    ea_st = _stats_call(edge_attr)

    xs1 = gather(x_enc, src2d)
    m1 = _conv_call(edge_attr, ea_st, ew, xs1, c1)
    agg1 = scatter(m1, dst2d)
    x2 = _upd_call(agg1, x_enc, p['c1_root'], p['c1_bias'][None, :])

    xs2 = gather(x2, src2d)
    m2 = _conv_call(edge_attr, ea_st, ew, xs2, c2)
    agg2 = scatter(m2, dst2d)
    x3 = _upd_call(agg2, x2, p['c2_root'], p['c2_bias'][None, :])

    xs3, xd3 = gather2(x3, src2d, dst2d)
    scores = _head_call(xs3, xd3, rw1[:H], rw1[H:], rs1, rb1, rw2, rs2, rb2,
                        p['r_w3'], p['r_b3'][None, :])
    gemb = _graph_call(x3, bat, gw,
                       p['d_w1'], p['d_b1'][None, :],
                       p['d_w2'], p['d_b2'][None, :])
    return (scores, gemb)


# speedup vs baseline: 2.9162x; 1.7084x over previous
import functools

import jax
import jax.numpy as jnp
from jax import lax
from jax.experimental import pallas as pl
from jax.experimental.pallas import tpu as pltpu
from jax.experimental.pallas import tpu_sc as plsc

N = 10000
E = 160000
H = 16
G = 64
EPS_BN = 1e-5
SLOPE = 0.01

NC = 2
NS = 16
NW = NC * NS
CH = 128
E_PAD = NW * 40 * CH
ROWS = E_PAD // CH
RPW = ROWS // NW
EPW = E_PAD // NW
BIN = N
NP = N + 16
OPW = N // NS

BE = 6400
GE = E // BE
BP = BE // 8
EP8 = E // 8
EPAD8 = E_PAD // 8
BS = BE // 128

_f32 = jnp.float32
_X_COLS = (0, 6, 7)
_EA_COLS = (0, 2, 7, 8, 9)


def _col_mask(cols):
    ci = lax.broadcasted_iota(jnp.int32, (1, H), 1)
    m = (ci == cols[0])
    for c in cols[1:]:
        m = m | (ci == c)
    return m.astype(_f32)


def _lrelu(z):
    return jnp.where(z >= 0, z, SLOPE * z)


def _dot(a, b):
    return jnp.dot(a, b, preferred_element_type=_f32)


def _xdot(a, b):
    return jnp.dot(a, b, preferred_element_type=_f32,
                   precision=jax.lax.Precision.HIGHEST)


def _enc2(z, w1, s1, b1, w2, s2, b2):
    h = _lrelu(_dot(z, w1) * s1 + b1)
    return _lrelu(_dot(h, w2) * s2 + b2)


def _ws(shape):
    return pl.BlockSpec(shape, lambda *_: tuple(0 for _ in shape))


def _pb():
    return pl.BlockSpec((BP, 128), lambda i: (i, 0))


def _norm_ab(mask_row, m, var):
    a = mask_row / (jnp.sqrt(var) + 1e-8) + (1.0 - mask_row)
    d = -m * a * mask_row
    return a, d


def _node_body(x_ref, w1, s1, b1, w2, s2, b2, o_ref):
    xv = x_ref[...]
    mask = _col_mask(_X_COLS)
    m = jnp.mean(xv, axis=0, keepdims=True)
    ssq = jnp.sum(xv * xv, axis=0, keepdims=True)
    var = (ssq - N * m * m) / (N - 1)
    a, d = _norm_ab(mask, m, var)
    xn = xv * a + d
    o_ref[...] = _enc2(xn, w1[...], s1[...], b1[...], w2[...], s2[...], b2[...])


def _stats_body(ea_ref, o_ref):
    @pl.when(pl.program_id(0) == 0)
    def _():
        o_ref[...] = jnp.zeros_like(o_ref)

    ev = ea_ref[...]
    o_ref[0:1, :] += jnp.sum(ev, axis=0, keepdims=True)
    o_ref[1:2, :] += jnp.sum(ev * ev, axis=0, keepdims=True)


def _fold_stats(st128):
    fi = lax.broadcasted_iota(jnp.int32, (128, H), 0)
    fj = lax.broadcasted_iota(jnp.int32, (128, H), 1)
    F = (fi % H == fj).astype(_f32)
    return _xdot(st128, F)


def _conv_body(ea_ref, st_ref, ew1, es1, eb1, ew2, es2, eb2, xs_ref,
               w1, s1, b1, w2c, s2c, b2c, bsel, o_ref):
    mask = _col_mask(_EA_COLS)
    st = _fold_stats(st_ref[...])
    mu = st[0:1, :] / E
    var = (st[1:2, :] - E * mu * mu) / (E - 1)
    a, d = _norm_ab(mask, mu, var)
    a8 = jnp.tile(a, (1, 8))
    d8 = jnp.tile(d, (1, 8))
    xn = ea_ref[...] * a8 + d8
    e1 = _lrelu(_dot(xn, ew1[...]) * es1[...] + eb1[...])
    ea_enc = _lrelu(_dot(e1, ew2[...]) * es2[...] + eb2[...])
    h = _lrelu(_dot(ea_enc, w1[...]) * s1[...] + b1[...])
    bf = jnp.bfloat16
    xsb = xs_ref[...].astype(bf)
    w2cv, s2cv, b2cv, bselv = w2c[...], s2c[...], b2c[...], bsel[...]
    acc = None
    for k in range(H):
        Wk = _lrelu(_dot(h, w2cv[k * 128:(k + 1) * 128, :])
                    * s2cv[k:k + 1, :] + b2cv[k:k + 1, :])
        xbk = jnp.dot(xsb, bselv[k * 128:(k + 1) * 128, :],
                      preferred_element_type=_f32)
        t = xbk * Wk.astype(bf).astype(_f32)
        acc = t if acc is None else acc + t
    o_ref[...] = acc


def _upd_body(agg_ref, x_ref, root, bias, o_ref):
    v = (agg_ref[0] + agg_ref[1] + bias[...] + _dot(x_ref[...], root[...]))
    n = jnp.sqrt(jnp.sum(v * v, axis=1, keepdims=True))
    o_ref[...] = v / jnp.maximum(n, 1e-12)


def _head_body(s_ref, d_ref, wa, wb, s1, b1, w2, s2, b2, w3, b3, o_ref):
    h = _lrelu((_dot(s_ref[...], wa[...]) + _dot(d_ref[...], wb[...]))
               * s1[...] + b1[...])
    h = _lrelu(_dot(h, w2[...]) * s2[...] + b2[...])
    o_ref[...] = _dot(h, w3[...]) + b3[...]


def _graph_body(x_ref, b_ref, gw1, gs1, gb1, gw2, gs2, gb2, dw1, db1, dw2,
                db2, o_ref):
    xv = x_ref[...]
    xg = _enc2(xv, gw1[...], gs1[...], gb1[...], gw2[...], gs2[...], gb2[...])
    bids = b_ref[...]
    gids = lax.broadcasted_iota(jnp.int32, (G, 1), 0)
    oh = (bids == gids).astype(jnp.bfloat16)
    xh = xg.astype(jnp.bfloat16)
    xl = (xg - xh.astype(_f32)).astype(jnp.bfloat16)
    sums = (jnp.dot(oh, xh, preferred_element_type=_f32)
            + jnp.dot(oh, xl, preferred_element_type=_f32))
    cnt = jnp.sum(oh.astype(_f32), axis=1, keepdims=True)
    mean = sums / jnp.maximum(cnt, 1.0)
    g1 = _lrelu(_dot(mean, dw1[...]) + db1[...])
    o_ref[...] = _dot(g1, dw2[...]) + db2[...]


def _sc_mesh():
    return plsc.VectorSubcoreMesh(core_axis_name="c", subcore_axis_name="s",
                                  num_cores=NC, num_subcores=NS)


def _gather_one(tbl, idx, out, idx_v, rows_v, sem):
    w = lax.axis_index("s") * NC + lax.axis_index("c")
    r0 = w * RPW
    pltpu.sync_copy(idx.at[pl.ds(r0, RPW)], idx_v)
    cps = [pltpu.async_copy(tbl.at[idx_v.at[j]],
                            rows_v.at[pl.ds(j * CH, CH)], sem)
           for j in range(RPW)]
    for cp in cps:
        cp.wait()
    pltpu.sync_copy(rows_v, out.at[pl.ds(r0 * CH, EPW)])


def _gather_body(tbl, idx, out, idx_v, rows_v, sem):
    _gather_one(tbl, idx, out, idx_v, rows_v, sem)


def _gather2_body(tbl, sidx, didx, outs, outd, idx_v, rows_v, sem):
    _gather_one(tbl, sidx, outs, idx_v, rows_v, sem)
    _gather_one(tbl, didx, outd, idx_v, rows_v, sem)


def _scatter_body(m, idx, out, idx_v, rows_v, acc_sh):
    c = lax.axis_index("c")
    s = lax.axis_index("s")
    w = s * NC + c

    def zbody(i, carry):
        rows_v[i] = jnp.zeros((H,), _f32)
        return carry

    lax.fori_loop(0, OPW, zbody, 0)
    pltpu.sync_copy(rows_v.at[pl.ds(0, OPW)], acc_sh.at[pl.ds(s * OPW, OPW)])
    plsc.subcore_barrier()
    pltpu.sync_copy(idx.at[pl.ds(w * RPW, RPW)], idx_v)
    pltpu.sync_copy(m.at[pl.ds(w * EPW, EPW)], rows_v)
    for j in range(RPW):
        pltpu.sync_copy(rows_v.at[pl.ds(j * CH, CH)], acc_sh.at[idx_v.at[j]],
                        add=True)
    plsc.subcore_barrier()
    pltpu.sync_copy(acc_sh.at[pl.ds(s * OPW, OPW)], rows_v.at[pl.ds(0, OPW)])
    pltpu.sync_copy(rows_v.at[pl.ds(0, OPW)], out.at[c, pl.ds(s * OPW, OPW)])


def _make_sc_calls():
    params = pltpu.CompilerParams(use_tc_tiling_on_sc=False)
    gather = pl.kernel(
        _gather_body,
        out_type=jax.ShapeDtypeStruct((E_PAD, H), _f32),
        mesh=_sc_mesh(),
        compiler_params=params,
        scratch_types=[pltpu.VMEM((RPW, CH), jnp.int32),
                       pltpu.VMEM((EPW, H), _f32),
                       pltpu.SemaphoreType.DMA],
    )
    gather2 = pl.kernel(
        _gather2_body,
        out_type=(jax.ShapeDtypeStruct((E_PAD, H), _f32),
                  jax.ShapeDtypeStruct((E_PAD, H), _f32)),
        mesh=_sc_mesh(),
        compiler_params=params,
        scratch_types=[pltpu.VMEM((RPW, CH), jnp.int32),
                       pltpu.VMEM((EPW, H), _f32),
                       pltpu.SemaphoreType.DMA],
    )
    scatter = pl.kernel(
        _scatter_body,
        out_type=jax.ShapeDtypeStruct((NC, N, H), _f32),
        mesh=_sc_mesh(),
        compiler_params=params,
        scratch_types=[pltpu.VMEM((RPW, CH), jnp.int32),
                       pltpu.VMEM((EPW, H), _f32),
                       pltpu.VMEM_SHARED((NP, H), _f32)],
    )
    return gather, gather2, scatter


def _node_call(x, ww):
    return pl.pallas_call(
        _node_body, grid=(1,),
        in_specs=[_ws((N, H)), _ws((H, H)), _ws((1, H)), _ws((1, H)),
                  _ws((H, H)), _ws((1, H)), _ws((1, H))],
        out_specs=_ws((N, H)),
        out_shape=jax.ShapeDtypeStruct((N, H), _f32),
    )(x, *ww)


def _stats_call(ea_p):
    return pl.pallas_call(
        _stats_body, grid=(GE,),
        in_specs=[_pb()],
        out_specs=_ws((2, 128)),
        out_shape=jax.ShapeDtypeStruct((2, 128), _f32),
    )(ea_p)


def _conv_call(ea_p, st, ew, xs_p, cw, bsel):
    return pl.pallas_call(
        _conv_body, grid=(GE,),
        in_specs=[_pb(), _ws((2, 128)),
                  _ws((128, 128)), _ws((1, 128)), _ws((1, 128)),
                  _ws((128, 128)), _ws((1, 128)), _ws((1, 128)),
                  _pb(),
                  _ws((128, 128)), _ws((1, 128)), _ws((1, 128)),
                  _ws((H * 128, 128)), _ws((H, 128)), _ws((H, 128)),
                  _ws((H * 128, 128))],
        out_specs=_pb(),
        out_shape=jax.ShapeDtypeStruct((EPAD8, 128), _f32),
    )(ea_p, st, *ew, xs_p, *cw, bsel)


def _upd_call(agg, x, root, bias):
    return pl.pallas_call(
        _upd_body, grid=(1,),
        in_specs=[pl.BlockSpec((NC, N, H), lambda i: (0, 0, 0)),
                  _ws((N, H)), _ws((H, H)), _ws((1, H))],
        out_specs=_ws((N, H)),
        out_shape=jax.ShapeDtypeStruct((N, H), _f32),
    )(agg, x, root, bias)


def _head_call(xs_p, xd_p, wa, wb, s1, b1, w2, s2, b2, w3, b3):
    return pl.pallas_call(
        _head_body, grid=(GE,),
        in_specs=[_pb(), _pb(), _ws((128, 128)), _ws((128, 128)),
                  _ws((1, 128)), _ws((1, 128)), _ws((128, 128)),
                  _ws((1, 128)), _ws((1, 128)),
                  _ws((128, 128)), _ws((1, 128))],
        out_specs=_pb(),
        out_shape=jax.ShapeDtypeStruct((EP8, 128), _f32),
    )(xs_p, xd_p, wa, wb, s1, b1, w2, s2, b2, w3, b3)


def _graph_call(x3, bat, gw, dw1, db1, dw2, db2):
    return pl.pallas_call(
        _graph_body, grid=(1,),
        in_specs=[_ws((N, H)), _ws((1, N)),
                  _ws((H, H)), _ws((1, H)), _ws((1, H)),
                  _ws((H, H)), _ws((1, H)), _ws((1, H)),
                  _ws((H, H)), _ws((1, H)), _ws((H, H)), _ws((1, H))],
        out_specs=_ws((G, H)),
        out_shape=jax.ShapeDtypeStruct((G, H), _f32),
    )(x3, bat, *gw, dw1, db1, dw2, db2)


def _fold(p, pre):
    s1 = p[pre + '_g1'] / jnp.sqrt(1.0 + EPS_BN)
    b1 = (p[pre + '_b1'] * s1 + p[pre + '_be1'])[None, :]
    s2 = p[pre + '_g2'] / jnp.sqrt(1.0 + EPS_BN)
    b2 = (p[pre + '_b2'] * s2 + p[pre + '_be2'])[None, :]
    return (p[pre + '_w1'], s1[None, :], b1,
            p[pre + '_w2'], s2[None, :], b2)


def _bd8(w):
    return jnp.kron(jnp.eye(8, dtype=w.dtype), w)


def _t8(v):
    return jnp.tile(v, (1, 8))


def _fold_packed(p, pre):
    w1, s1, b1, w2, s2, b2 = _fold(p, pre)
    return (_bd8(w1), _t8(s1), _t8(b1), _bd8(w2), _t8(s2), _t8(b2))


def _fold_conv(p, pre):
    w1, s1, b1, w2, s2, b2 = _fold(p, pre)
    w2c = jnp.concatenate([_bd8(w2[:, k * H:(k + 1) * H]) for k in range(H)],
                          axis=0)
    s2c = jnp.concatenate([_t8(s2[:, k * H:(k + 1) * H]) for k in range(H)],
                          axis=0)
    b2c = jnp.concatenate([_t8(b2[:, k * H:(k + 1) * H]) for k in range(H)],
                          axis=0)
    return (_bd8(w1), _t8(s1), _t8(b1), w2c, s2c, b2c)


def _bsel():
    rk = jnp.arange(H * 128)
    k = rk // 128
    pp = rk % 128
    q = jnp.arange(128)
    m = (pp % H == k)[:, None] & ((pp // H)[:, None] == (q // H)[None, :])
    return m.astype(jnp.bfloat16)


def kernel(x, edge_index, edge_attr, batch, params):
    p = params
    src = edge_index[0]
    dst = edge_index[1]
    padn = E_PAD - E
    src2d = jnp.concatenate(
        [src, jnp.zeros((padn,), jnp.int32)]).reshape(ROWS, CH)
    dst2d = jnp.concatenate(
        [dst, jnp.full((padn,), BIN, jnp.int32)]).reshape(ROWS, CH)
    bat = batch.reshape(1, N)

    gather, gather2, scatter = _make_sc_calls()

    ne = _fold(p, 'ne')
    ew = _fold_packed(p, 'ee')
    c1 = _fold_conv(p, 'c1')
    c2 = _fold_conv(p, 'c2')
    bsel = _bsel()
    rw1, rs1, rb1, rw2, rs2, rb2 = _fold(p, 'r')
    gw = _fold(p, 'g')
    ha = (_bd8(rw1[:H]), _bd8(rw1[H:]), _t8(rs1), _t8(rb1), _bd8(rw2),
          _t8(rs2), _t8(rb2),
          jnp.kron(jnp.eye(8, dtype=_f32), p['r_w3'] @ jnp.ones((1, H), _f32)),
          jnp.tile(p['r_b3'][None, :], (1, 128)))

    ea_p = edge_attr.reshape(EP8, 128)

    x_enc = _node_call(x, ne)
    ea_st = _stats_call(ea_p)

    xs1 = gather(x_enc, src2d).reshape(EPAD8, 128)
    m1 = _conv_call(ea_p, ea_st, ew, xs1, c1, bsel).reshape(E_PAD, H)
    agg1 = scatter(m1, dst2d)
    x2 = _upd_call(agg1, x_enc, p['c1_root'], p['c1_bias'][None, :])

    xs2 = gather(x2, src2d).reshape(EPAD8, 128)
    m2 = _conv_call(ea_p, ea_st, ew, xs2, c2, bsel).reshape(E_PAD, H)
    agg2 = scatter(m2, dst2d)
    x3 = _upd_call(agg2, x2, p['c2_root'], p['c2_bias'][None, :])

    xs3, xd3 = gather2(x3, src2d, dst2d)
    scores_dup = _head_call(xs3.reshape(EPAD8, 128), xd3.reshape(EPAD8, 128),
                            *ha)
    scores = scores_dup.reshape(E, H)[:, :1]
    gemb = _graph_call(x3, bat, gw,
                       p['d_w1'], p['d_b1'][None, :],
                       p['d_w2'], p['d_b2'][None, :])
    return (scores, gemb)
